# baseline (speedup 1.0000x reference)
import jax
import jax.numpy as jnp
from jax.experimental import pallas as pl


def _conv_matrix(W):
    Wf = W.reshape(3, 3)
    p = jnp.arange(16)
    h, w = p // 4, p % 4
    dh = h[None, :] - h[:, None]
    dw = w[None, :] - w[:, None]
    valid = (jnp.abs(dh) <= 1) & (jnp.abs(dw) <= 1)
    M = jnp.where(valid, Wf[jnp.clip(dh + 1, 0, 2), jnp.clip(dw + 1, 0, 2)], 0.0)
    return M


def _body(x_ref, m_ref, o_ref):
    xb = x_ref[...]
    y = jnp.dot(xb, m_ref[...], preferred_element_type=jnp.float32)
    o_ref[...] = jnp.where(xb != 0.0, y, 0.0)


def kernel(x, W):
    N = x.shape[0]
    xf = x.reshape(N // 8, 128)
    M = _conv_matrix(W)
    B = jnp.kron(jnp.eye(8, dtype=jnp.float32), M.T)

    BLK = 1024
    rows = xf.shape[0]
    out = pl.pallas_call(
        _body,
        grid=(rows // BLK,),
        in_specs=[
            pl.BlockSpec((BLK, 128), lambda i: (i, 0)),
            pl.BlockSpec((128, 128), lambda i: (0, 0)),
        ],
        out_specs=pl.BlockSpec((BLK, 128), lambda i: (i, 0)),
        out_shape=jax.ShapeDtypeStruct((rows, 128), jnp.float32),
    )(xf, B)
    return out.reshape(x.shape)


# speedup vs baseline: 91.7677x; 91.7677x over previous
import jax
import jax.numpy as jnp
from jax.experimental import pallas as pl
from jax.experimental.pallas import tpu as pltpu


def _conv_matrix(W):
    Wf = W.reshape(3, 3)
    p = jnp.arange(16)
    h, w = p // 4, p % 4
    dh = h[None, :] - h[:, None]
    dw = w[None, :] - w[:, None]
    valid = (jnp.abs(dh) <= 1) & (jnp.abs(dw) <= 1)
    return jnp.where(valid, Wf[jnp.clip(dh + 1, 0, 2), jnp.clip(dw + 1, 0, 2)], 0.0)


def _body(x_ref, m_ref, o_ref):
    for p in range(16):
        h, w = divmod(p, 4)
        acc = None
        for q in range(16):
            h2, w2 = divmod(q, 4)
            if abs(h2 - h) <= 1 and abs(w2 - w) <= 1:
                t = x_ref[q] * m_ref[p, q]
                acc = t if acc is None else acc + t
        o_ref[p] = jnp.where(x_ref[p] != 0.0, acc, 0.0)


def kernel(x, W):
    N = x.shape[0]
    xt = x.transpose(1, 2, 3, 0).reshape(16, N // 128, 128)
    M = _conv_matrix(W)

    BM = 64
    rows = N // 128
    out = pl.pallas_call(
        _body,
        grid=(rows // BM,),
        in_specs=[
            pl.BlockSpec((16, BM, 128), lambda i: (0, i, 0)),
            pl.BlockSpec(memory_space=pltpu.SMEM),
        ],
        out_specs=pl.BlockSpec((16, BM, 128), lambda i: (0, i, 0)),
        out_shape=jax.ShapeDtypeStruct((16, rows, 128), jnp.float32),
    )(xt, M)
    return out.reshape(4, 4, 1, N).transpose(3, 0, 1, 2)


# speedup vs baseline: 162.0476x; 1.7658x over previous
import jax
import jax.numpy as jnp
from jax.experimental import pallas as pl
from jax.experimental.pallas import tpu as pltpu


def _conv_matrix(W):
    Wf = W.reshape(3, 3)
    p = jnp.arange(16)
    h, w = p // 4, p % 4
    dh = h[None, :] - h[:, None]
    dw = w[None, :] - w[:, None]
    valid = (jnp.abs(dh) <= 1) & (jnp.abs(dw) <= 1)
    return jnp.where(valid, Wf[jnp.clip(dh + 1, 0, 2), jnp.clip(dw + 1, 0, 2)], 0.0)


def _body(x_ref, m_ref, o_ref):
    for p in range(16):
        h, w = divmod(p, 4)
        acc = None
        for q in range(16):
            h2, w2 = divmod(q, 4)
            if abs(h2 - h) <= 1 and abs(w2 - w) <= 1:
                t = x_ref[q] * m_ref[p, q]
                acc = t if acc is None else acc + t
        o_ref[p] = jnp.where(x_ref[p] != 0.0, acc, 0.0)


def kernel(x, W):
    N = x.shape[0]
    xt = x.transpose(1, 2, 3, 0).reshape(16, N // 128, 128)
    M = _conv_matrix(W)

    BM = 256
    rows = N // 128
    out = pl.pallas_call(
        _body,
        grid=(rows // BM,),
        in_specs=[
            pl.BlockSpec((16, BM, 128), lambda i: (0, i, 0)),
            pl.BlockSpec(memory_space=pltpu.SMEM),
        ],
        out_specs=pl.BlockSpec((16, BM, 128), lambda i: (0, i, 0)),
        out_shape=jax.ShapeDtypeStruct((16, rows, 128), jnp.float32),
    )(xt, M)
    return out.reshape(4, 4, 1, N).transpose(3, 0, 1, 2)


# speedup vs baseline: 184.9079x; 1.1411x over previous
import jax
import jax.numpy as jnp
from jax.experimental import pallas as pl
from jax.experimental.pallas import tpu as pltpu


def _conv_matrix(W):
    Wf = W.reshape(3, 3)
    p = jnp.arange(16)
    h, w = p // 4, p % 4
    dh = h[None, :] - h[:, None]
    dw = w[None, :] - w[:, None]
    valid = (jnp.abs(dh) <= 1) & (jnp.abs(dw) <= 1)
    return jnp.where(valid, Wf[jnp.clip(dh + 1, 0, 2), jnp.clip(dw + 1, 0, 2)], 0.0)


def _body(x_ref, m_ref, o_ref):
    for p in range(16):
        h, w = divmod(p, 4)
        acc = None
        for q in range(16):
            h2, w2 = divmod(q, 4)
            if abs(h2 - h) <= 1 and abs(w2 - w) <= 1:
                t = x_ref[q] * m_ref[p, q]
                acc = t if acc is None else acc + t
        o_ref[p] = jnp.where(x_ref[p] != 0.0, acc, 0.0)


def kernel(x, W):
    N = x.shape[0]
    xt = x.transpose(1, 2, 3, 0).reshape(16, N // 128, 128)
    M = _conv_matrix(W)

    BM = 512
    rows = N // 128
    out = pl.pallas_call(
        _body,
        grid=(rows // BM,),
        in_specs=[
            pl.BlockSpec((16, BM, 128), lambda i: (0, i, 0)),
            pl.BlockSpec(memory_space=pltpu.SMEM),
        ],
        out_specs=pl.BlockSpec((16, BM, 128), lambda i: (0, i, 0)),
        out_shape=jax.ShapeDtypeStruct((16, rows, 128), jnp.float32),
    )(xt, M)
    return out.reshape(4, 4, 1, N).transpose(3, 0, 1, 2)


# speedup vs baseline: 194.0895x; 1.0497x over previous
import jax
import jax.numpy as jnp
from jax.experimental import pallas as pl
from jax.experimental.pallas import tpu as pltpu


def _conv_matrix(W):
    Wf = W.reshape(3, 3)
    p = jnp.arange(16)
    h, w = p // 4, p % 4
    dh = h[None, :] - h[:, None]
    dw = w[None, :] - w[:, None]
    valid = (jnp.abs(dh) <= 1) & (jnp.abs(dw) <= 1)
    return jnp.where(valid, Wf[jnp.clip(dh + 1, 0, 2), jnp.clip(dw + 1, 0, 2)], 0.0)


def _body(x_ref, m_ref, o_ref):
    for p in range(16):
        h, w = divmod(p, 4)
        acc = None
        for q in range(16):
            h2, w2 = divmod(q, 4)
            if abs(h2 - h) <= 1 and abs(w2 - w) <= 1:
                t = x_ref[q] * m_ref[p, q]
                acc = t if acc is None else acc + t
        o_ref[p] = jnp.where(x_ref[p] != 0.0, acc, 0.0)


def kernel(x, W):
    N = x.shape[0]
    xt = x.transpose(1, 2, 3, 0).reshape(16, N // 128, 128)
    M = _conv_matrix(W)

    BM = 1024
    rows = N // 128
    out = pl.pallas_call(
        _body,
        grid=(rows // BM,),
        in_specs=[
            pl.BlockSpec((16, BM, 128), lambda i: (0, i, 0)),
            pl.BlockSpec(memory_space=pltpu.SMEM),
        ],
        out_specs=pl.BlockSpec((16, BM, 128), lambda i: (0, i, 0)),
        out_shape=jax.ShapeDtypeStruct((16, rows, 128), jnp.float32),
    )(xt, M)
    return out.reshape(4, 4, 1, N).transpose(3, 0, 1, 2)
